# baseline (speedup 1.0000x reference)
import functools

import jax
import jax.numpy as jnp
from jax import lax
from jax.experimental import pallas as pl
from jax.experimental.pallas import tpu as pltpu
from jax.experimental.pallas import tpu_sc as plsc

_INFO = plsc.get_sparse_core_info()
_NC = _INFO.num_cores
_NS = _INFO.num_subcores
_NW = _NC * _NS
_L = _INFO.num_lanes

_BATCH = 16384
_BPW = _BATCH // _NW
_CHUNK = 128
_NCHUNK = _BPW // _CHUNK
_NCOL = 10


def _body(idx_hbm, *refs):
    col_hbm = refs[:_NCOL]
    out_hbm = refs[_NCOL]
    idx_v, cols_v, stage_v, sem = refs[_NCOL + 1:]

    wid = lax.axis_index("s") * _NC + lax.axis_index("c")
    base = wid * _BPW

    pltpu.sync_copy(idx_hbm.at[pl.ds(wid * _NCHUNK, _NCHUNK)], idx_v)

    copies = []
    for c in range(_NCOL):
        for k in range(_NCHUNK):
            copies.append(pltpu.async_copy(
                col_hbm[c].at[idx_v.at[k]],
                cols_v.at[c].at[pl.ds(k * _CHUNK, _CHUNK)],
                sem))
    for cp in copies:
        cp.wait()

    lanes = jnp.arange(_L, dtype=jnp.int32)

    def interleave(g, _):
        rows = g * _L + lanes
        for c in range(_NCOL):
            vals = cols_v.at[c][pl.ds(g * _L, _L)]
            ocol = jnp.full((_L,), c, dtype=jnp.int32)
            plsc.store_scatter(stage_v, [rows, ocol], vals)
        return 0

    lax.fori_loop(0, _BPW // _L, interleave, 0)

    pltpu.sync_copy(stage_v, out_hbm.at[pl.ds(base, _BPW)])


@jax.jit
def _run(camera_idx, intrinsic_deltas, rotation_deltas, translation_deltas):
    cols = tuple(intrinsic_deltas[:, c] for c in range(4))
    cols += tuple(rotation_deltas[:, c] for c in range(3))
    cols += tuple(translation_deltas[:, c] for c in range(3))

    mesh = plsc.VectorSubcoreMesh(core_axis_name="c", subcore_axis_name="s")
    kfn = functools.partial(
        pl.kernel,
        out_type=jax.ShapeDtypeStruct((_BATCH, _NCOL), jnp.float32),
        mesh=mesh,
        scratch_types=[
            pltpu.VMEM((_NCHUNK, _CHUNK), jnp.int32),
            pltpu.VMEM((_NCOL, _BPW), jnp.float32),
            pltpu.VMEM((_BPW, _NCOL), jnp.float32),
            pltpu.SemaphoreType.DMA,
        ],
        compiler_params=pltpu.CompilerParams(
            use_tc_tiling_on_sc=False, needs_layout_passes=False),
    )(_body)
    idx2 = camera_idx.reshape(_NW * _NCHUNK, _CHUNK)
    return kfn(idx2, *cols)


def kernel(camera_idx, intrinsic_deltas, rotation_deltas, translation_deltas):
    return _run(camera_idx.astype(jnp.int32), intrinsic_deltas,
                rotation_deltas, translation_deltas)


# speedup vs baseline: 1.3236x; 1.3236x over previous
import functools

import jax
import jax.numpy as jnp
from jax import lax
from jax.experimental import pallas as pl
from jax.experimental.pallas import tpu as pltpu
from jax.experimental.pallas import tpu_sc as plsc

_INFO = plsc.get_sparse_core_info()
_NC = _INFO.num_cores
_NS = _INFO.num_subcores
_NW = _NC * _NS
_L = _INFO.num_lanes

_BATCH = 16384
_BPW = _BATCH // _NW
_CHUNK = 128
_NCHUNK = _BPW // _CHUNK
_NCOL = 10


def _body(idx_hbm, t4_hbm, t3a_hbm, t3b_hbm, out_hbm,
          idx_v, cols_v, stage_v, sem):
    wid = lax.axis_index("s") * _NC + lax.axis_index("c")
    base = wid * _BPW

    pltpu.sync_copy(idx_hbm.at[pl.ds(wid * _NCHUNK, _NCHUNK)], idx_v)

    srcs = ([t4_hbm.at[c] for c in range(4)]
            + [t3a_hbm.at[c] for c in range(3)]
            + [t3b_hbm.at[c] for c in range(3)])
    copies = []
    for c in range(_NCOL):
        for k in range(_NCHUNK):
            copies.append(pltpu.async_copy(
                srcs[c].at[idx_v.at[k]],
                cols_v.at[c].at[pl.ds(k * _CHUNK, _CHUNK)],
                sem))
    for cp in copies:
        cp.wait()

    lanes = jnp.arange(_L, dtype=jnp.int32)

    def interleave(g, _):
        rows = g * _L + lanes
        for c in range(_NCOL):
            vals = cols_v.at[c][pl.ds(g * _L, _L)]
            ocol = jnp.full((_L,), c, dtype=jnp.int32)
            plsc.store_scatter(stage_v, [rows, ocol], vals)
        return 0

    lax.fori_loop(0, _BPW // _L, interleave, 0)

    pltpu.sync_copy(stage_v, out_hbm.at[pl.ds(base, _BPW)])


@jax.jit
def _run(camera_idx, intrinsic_deltas, rotation_deltas, translation_deltas):
    mesh = plsc.VectorSubcoreMesh(core_axis_name="c", subcore_axis_name="s")
    kfn = functools.partial(
        pl.kernel,
        out_type=jax.ShapeDtypeStruct((_BATCH, _NCOL), jnp.float32),
        mesh=mesh,
        scratch_types=[
            pltpu.VMEM((_NCHUNK, _CHUNK), jnp.int32),
            pltpu.VMEM((_NCOL, _BPW), jnp.float32),
            pltpu.VMEM((_BPW, _NCOL), jnp.float32),
            pltpu.SemaphoreType.DMA,
        ],
        compiler_params=pltpu.CompilerParams(
            use_tc_tiling_on_sc=False, needs_layout_passes=False),
    )(_body)
    idx2 = camera_idx.reshape(_NW * _NCHUNK, _CHUNK)
    return kfn(idx2, intrinsic_deltas.T, rotation_deltas.T,
               translation_deltas.T)


def kernel(camera_idx, intrinsic_deltas, rotation_deltas, translation_deltas):
    return _run(camera_idx.astype(jnp.int32), intrinsic_deltas,
                rotation_deltas, translation_deltas)
